# baseline (speedup 1.0000x reference)
import functools

import jax
import jax.numpy as jnp
from jax import lax
from jax.experimental import pallas as pl
from jax.experimental.pallas import tpu as pltpu
from jax.experimental.pallas import tpu_sc as plsc

ROWS = 64
N = 32768
TOPK = 64
L = 16
NVREG = N // L
CHUNKS = 64
VPC = NVREG // CHUNKS
CAP = 192
NW = 32
ROWS_PER_W = ROWS // NW

_NEG = float("-inf")
_BIGI = 0x7FFFFFFF

_GDN = lax.GatherDimensionNumbers(
    offset_dims=(), collapsed_slice_dims=(0,), start_index_map=(0,))


def _shuf(v, perm):
    return lax.gather(v, perm[:, None], _GDN, (1,),
                      mode=lax.GatherScatterMode.PROMISE_IN_BOUNDS)


def _bfly(v, op, lanes):
    for s in (8, 4, 2, 1):
        v = op(v, _shuf(v, lanes ^ s))
    return v


@functools.partial(
    pl.kernel,
    out_type=(
        jax.ShapeDtypeStruct((ROWS, TOPK), jnp.float32),
        jax.ShapeDtypeStruct((ROWS, TOPK), jnp.int32),
    ),
    mesh=plsc.VectorSubcoreMesh(core_axis_name="c", subcore_axis_name="s"),
    compiler_params=pltpu.CompilerParams(needs_layout_passes=False),
    scratch_types=[
        pltpu.VMEM((N,), jnp.float32),
        pltpu.VMEM((N,), jnp.float32),
        pltpu.VMEM((CAP * L,), jnp.float32),
        pltpu.VMEM((CAP * L,), jnp.int32),
        pltpu.VMEM((TOPK,), jnp.float32),
        pltpu.VMEM((TOPK,), jnp.int32),
        pltpu.SemaphoreType.DMA,
        pltpu.SemaphoreType.DMA,
    ],
)
def _topk_kernel(inp_hbm, vals_hbm, idxs_hbm, row_a, row_b, cval, cidx,
                 oval, oidx, sem_a, sem_b):
    wid = lax.axis_index("c") * 16 + lax.axis_index("s")
    lanes = lax.iota(jnp.int32, L)
    neg = jnp.full((L,), _NEG, jnp.float32)
    bigi = jnp.full((L,), _BIGI, jnp.int32)

    rows = (row_a, row_b)
    copies = [
        pltpu.async_copy(inp_hbm.at[rr * NW + wid], rows[rr], (sem_a, sem_b)[rr])
        for rr in range(ROWS_PER_W)
    ]

    for rr in range(ROWS_PER_W):
        r = rr * NW + wid
        row_v = rows[rr]
        copies[rr].wait()

        mg = []
        for g in range(4):
            def gq(t, ms, g=g):
                m0, m1, m2, m3 = ms
                base = (g * (NVREG // 4) + t * 4) * L
                m0 = jnp.maximum(m0, row_v[pl.ds(base, L)])
                m1 = jnp.maximum(m1, row_v[pl.ds(base + L, L)])
                m2 = jnp.maximum(m2, row_v[pl.ds(base + 2 * L, L)])
                m3 = jnp.maximum(m3, row_v[pl.ds(base + 3 * L, L)])
                return (m0, m1, m2, m3)

            ms = lax.fori_loop(0, NVREG // 16, gq, (neg, neg, neg, neg))
            mg.append(jnp.maximum(jnp.maximum(ms[0], ms[1]),
                                  jnp.maximum(ms[2], ms[3])))
        mn = jnp.minimum(jnp.minimum(mg[0], mg[1]),
                         jnp.minimum(mg[2], mg[3]))
        t0v = _bfly(mn, jnp.minimum, lanes)

        clampv = jnp.full((L,), (CAP - 1) * L, jnp.int32) + lanes
        c16 = jnp.full((L,), L, jnp.int32)
        zero = jnp.zeros((L,), jnp.int32)

        def collect(g, st):
            ptr16, iv = st
            for j in range(4):
                v = row_v[pl.ds((g * 4 + j) * L, L)]
                msk = v >= t0v
                pos = jnp.minimum(ptr16, clampv)
                plsc.store_scatter(cidx, [pos], iv, mask=msk)
                ptr16 = ptr16 + jnp.where(msk, c16, zero)
                iv = iv + c16
            return ptr16, iv

        ptr16, _ = lax.fori_loop(0, NVREG // 4, collect, (lanes, lanes))
        maxd = jnp.minimum(
            lax.shift_right_logical(_bfly(ptr16, jnp.maximum, lanes)[0], 4),
            CAP - 1)
        cntd = lax.shift_right_logical(ptr16 - lanes, 4)

        def mat_body(d, carry):
            dv = jnp.full((L,), d, jnp.int32)
            valid = dv < cntd
            idxv = jnp.where(valid, cidx[pl.ds(d * L, L)], zero)
            vv = plsc.load_gather(row_v, [idxv])
            cval[pl.ds(d * L, L)] = jnp.where(valid, vv, neg)
            return carry

        lax.fori_loop(0, maxd + 1, mat_body, 0)

        def isort_body(i, carry):
            kv = cval[pl.ds(i * L, L)]
            ki = cidx[pl.ds(i * L, L)]

            def down(jj, moving):
                j = i - 1 - jj
                vj = cval[pl.ds(j * L, L)]
                ij = cidx[pl.ds(j * L, L)]
                lt = (kv > vj) | ((kv == vj) & (ki < ij))
                pos1 = (j + 1) * L + lanes
                wv = jnp.where(lt, vj, kv)
                wi = jnp.where(lt, ij, ki)
                plsc.store_scatter(cval, [pos1], wv, mask=moving)
                plsc.store_scatter(cidx, [pos1], wi, mask=moving)
                return moving & lt

            moving = lax.fori_loop(0, i, down, jnp.ones((L,), jnp.bool_))
            plsc.store_scatter(cval, [lanes], kv, mask=moving)
            plsc.store_scatter(cidx, [lanes], ki, mask=moving)
            return carry

        lax.fori_loop(1, maxd, isort_body, 0)

        def merge_round(k, hp):
            addr = hp * L + lanes
            hv = plsc.load_gather(cval, [addr])
            hi = plsc.load_gather(cidx, [addr])
            mv = _bfly(hv, jnp.maximum, lanes)
            lm = hv == mv
            civ = _bfly(jnp.where(lm, hi, bigi), jnp.minimum, lanes)
            chosen = lm & (hi == civ)
            l0 = lanes == 0
            kv = jnp.full((L,), k, jnp.int32)
            plsc.store_scatter(oval, [kv], mv, mask=l0)
            plsc.store_scatter(oidx, [kv], civ, mask=l0)
            return hp + chosen.astype(jnp.int32)

        lax.fori_loop(0, TOPK, merge_round, jnp.zeros((L,), jnp.int32))

        pltpu.sync_copy(oval, vals_hbm.at[r])
        pltpu.sync_copy(oidx, idxs_hbm.at[r])


def kernel(inp, shared_refpanel):
    vals, idxs = _topk_kernel(inp)
    return (vals, idxs)


# speedup vs baseline: 1.1046x; 1.1046x over previous
import functools

import jax
import jax.numpy as jnp
from jax import lax
from jax.experimental import pallas as pl
from jax.experimental.pallas import tpu as pltpu
from jax.experimental.pallas import tpu_sc as plsc

ROWS = 64
N = 32768
TOPK = 64
L = 16
NVREG = N // L
CHUNKS = 64
VPC = NVREG // CHUNKS
CAP = 192
NW = 32
ROWS_PER_W = ROWS // NW

_NEG = float("-inf")
_BIGI = 0x7FFFFFFF

_GDN = lax.GatherDimensionNumbers(
    offset_dims=(), collapsed_slice_dims=(0,), start_index_map=(0,))


def _shuf(v, perm):
    return lax.gather(v, perm[:, None], _GDN, (1,),
                      mode=lax.GatherScatterMode.PROMISE_IN_BOUNDS)


def _bfly(v, op, lanes):
    for s in (8, 4, 2, 1):
        v = op(v, _shuf(v, lanes ^ s))
    return v


@functools.partial(
    pl.kernel,
    out_type=(
        jax.ShapeDtypeStruct((ROWS, TOPK), jnp.float32),
        jax.ShapeDtypeStruct((ROWS, TOPK), jnp.int32),
    ),
    mesh=plsc.VectorSubcoreMesh(core_axis_name="c", subcore_axis_name="s"),
    compiler_params=pltpu.CompilerParams(needs_layout_passes=False),
    scratch_types=[
        pltpu.VMEM((N,), jnp.float32),
        pltpu.VMEM((N,), jnp.float32),
        pltpu.VMEM((CAP * L,), jnp.float32),
        pltpu.VMEM((CAP * L,), jnp.int32),
        pltpu.VMEM((TOPK,), jnp.float32),
        pltpu.VMEM((TOPK,), jnp.int32),
        pltpu.SemaphoreType.DMA,
        pltpu.SemaphoreType.DMA,
    ],
)
def _topk_kernel(inp_hbm, vals_hbm, idxs_hbm, row_a, row_b, cval, cidx,
                 oval, oidx, sem_a, sem_b):
    wid = lax.axis_index("c") * 16 + lax.axis_index("s")
    lanes = lax.iota(jnp.int32, L)
    neg = jnp.full((L,), _NEG, jnp.float32)
    bigi = jnp.full((L,), _BIGI, jnp.int32)

    rows = (row_a, row_b)
    copies = [
        pltpu.async_copy(inp_hbm.at[rr * NW + wid], rows[rr], (sem_a, sem_b)[rr])
        for rr in range(ROWS_PER_W)
    ]

    for rr in range(ROWS_PER_W):
        r = rr * NW + wid
        row_v = rows[rr]
        copies[rr].wait()

        def chunk_body(c, t0v):
            m = [neg, neg, neg, neg]
            for j in range(VPC):
                m[j % 4] = jnp.maximum(
                    m[j % 4], row_v[pl.ds(c * (VPC * L) + j * L, L)])
            mm = jnp.maximum(jnp.maximum(m[0], m[1]),
                             jnp.maximum(m[2], m[3]))
            return jnp.minimum(t0v, _bfly(mm, jnp.maximum, lanes))

        t0v = lax.fori_loop(
            0, CHUNKS, chunk_body, jnp.full((L,), float("inf"), jnp.float32))

        clampv = jnp.full((L,), (CAP - 1) * L, jnp.int32) + lanes
        c16 = jnp.full((L,), L, jnp.int32)
        zero = jnp.zeros((L,), jnp.int32)

        def collect(g, st):
            ptr16, iv = st
            for j in range(4):
                v = row_v[pl.ds((g * 4 + j) * L, L)]
                msk = v >= t0v
                pos = jnp.minimum(ptr16, clampv)
                plsc.store_scatter(cidx, [pos], iv, mask=msk)
                ptr16 = ptr16 + jnp.where(msk, c16, zero)
                iv = iv + c16
            return ptr16, iv

        ptr16, _ = lax.fori_loop(0, NVREG // 4, collect, (lanes, lanes))
        maxd = jnp.minimum(
            lax.shift_right_logical(_bfly(ptr16, jnp.maximum, lanes)[0], 4),
            CAP - 1)
        cntd = lax.shift_right_logical(ptr16 - lanes, 4)

        def mat_body(d, carry):
            dv = jnp.full((L,), d, jnp.int32)
            valid = dv < cntd
            idxv = jnp.where(valid, cidx[pl.ds(d * L, L)], zero)
            vv = plsc.load_gather(row_v, [idxv])
            cval[pl.ds(d * L, L)] = jnp.where(valid, vv, neg)
            return carry

        lax.fori_loop(0, maxd + 1, mat_body, 0)

        def isort_body(i, carry):
            kv = cval[pl.ds(i * L, L)]
            ki = cidx[pl.ds(i * L, L)]

            def down(jj, moving):
                j = i - 1 - jj
                vj = cval[pl.ds(j * L, L)]
                ij = cidx[pl.ds(j * L, L)]
                lt = (kv > vj) | ((kv == vj) & (ki < ij))
                pos1 = (j + 1) * L + lanes
                wv = jnp.where(lt, vj, kv)
                wi = jnp.where(lt, ij, ki)
                plsc.store_scatter(cval, [pos1], wv, mask=moving)
                plsc.store_scatter(cidx, [pos1], wi, mask=moving)
                return moving & lt

            moving = lax.fori_loop(0, i, down, jnp.ones((L,), jnp.bool_))
            plsc.store_scatter(cval, [lanes], kv, mask=moving)
            plsc.store_scatter(cidx, [lanes], ki, mask=moving)
            return carry

        lax.fori_loop(1, maxd, isort_body, 0)

        def merge_round(k, hp):
            addr = hp * L + lanes
            hv = plsc.load_gather(cval, [addr])
            hi = plsc.load_gather(cidx, [addr])
            mv = _bfly(hv, jnp.maximum, lanes)
            lm = hv == mv
            civ = _bfly(jnp.where(lm, hi, bigi), jnp.minimum, lanes)
            chosen = lm & (hi == civ)
            l0 = lanes == 0
            kv = jnp.full((L,), k, jnp.int32)
            plsc.store_scatter(oval, [kv], mv, mask=l0)
            plsc.store_scatter(oidx, [kv], civ, mask=l0)
            return hp + chosen.astype(jnp.int32)

        lax.fori_loop(0, TOPK, merge_round, jnp.zeros((L,), jnp.int32))

        pltpu.sync_copy(oval, vals_hbm.at[r])
        pltpu.sync_copy(oidx, idxs_hbm.at[r])


def kernel(inp, shared_refpanel):
    vals, idxs = _topk_kernel(inp)
    return (vals, idxs)


# speedup vs baseline: 1.7030x; 1.5417x over previous
import functools

import jax
import jax.numpy as jnp
from jax import lax
from jax.experimental import pallas as pl
from jax.experimental.pallas import tpu as pltpu
from jax.experimental.pallas import tpu_sc as plsc

ROWS = 64
N = 32768
TOPK = 64
L = 16
NVREG = N // L
CHUNKS = 64
VPC = NVREG // CHUNKS
CAP = 192
NW = 32
ROWS_PER_W = ROWS // NW

_NEG = float("-inf")
_BIGI = 0x7FFFFFFF

_GDN = lax.GatherDimensionNumbers(
    offset_dims=(), collapsed_slice_dims=(0,), start_index_map=(0,))


def _shuf(v, perm):
    return lax.gather(v, perm[:, None], _GDN, (1,),
                      mode=lax.GatherScatterMode.PROMISE_IN_BOUNDS)


def _bfly(v, op, lanes):
    for s in (8, 4, 2, 1):
        v = op(v, _shuf(v, lanes ^ s))
    return v


@functools.partial(
    pl.kernel,
    out_type=(
        jax.ShapeDtypeStruct((ROWS, TOPK), jnp.float32),
        jax.ShapeDtypeStruct((ROWS, TOPK), jnp.int32),
    ),
    mesh=plsc.VectorSubcoreMesh(core_axis_name="c", subcore_axis_name="s"),
    compiler_params=pltpu.CompilerParams(needs_layout_passes=False),
    scratch_types=[
        pltpu.VMEM((N,), jnp.float32),
        pltpu.VMEM((N,), jnp.float32),
        pltpu.VMEM((CAP * L,), jnp.float32),
        pltpu.VMEM((CAP * L,), jnp.int32),
        pltpu.VMEM((TOPK,), jnp.float32),
        pltpu.VMEM((TOPK,), jnp.int32),
        pltpu.SemaphoreType.DMA,
        pltpu.SemaphoreType.DMA,
    ],
)
def _topk_kernel(inp_hbm, vals_hbm, idxs_hbm, row_a, row_b, cval, cidx,
                 oval, oidx, sem_a, sem_b):
    wid = lax.axis_index("c") * 16 + lax.axis_index("s")
    lanes = lax.iota(jnp.int32, L)
    neg = jnp.full((L,), _NEG, jnp.float32)
    bigi = jnp.full((L,), _BIGI, jnp.int32)

    rows = (row_a, row_b)
    copies = [
        pltpu.async_copy(inp_hbm.at[rr * NW + wid], rows[rr], (sem_a, sem_b)[rr])
        for rr in range(ROWS_PER_W)
    ]

    for rr in range(ROWS_PER_W):
        r = rr * NW + wid
        row_v = rows[rr]
        copies[rr].wait()

        def chunk_body(c, t0v):
            m = [neg, neg, neg, neg]
            for j in range(VPC):
                m[j % 4] = jnp.maximum(
                    m[j % 4], row_v[pl.ds(c * (VPC * L) + j * L, L)])
            mm = jnp.maximum(jnp.maximum(m[0], m[1]),
                             jnp.maximum(m[2], m[3]))
            return jnp.minimum(t0v, _bfly(mm, jnp.maximum, lanes))

        t0v = lax.fori_loop(
            0, CHUNKS, chunk_body, jnp.full((L,), float("inf"), jnp.float32))

        clampv = jnp.full((L,), (CAP - 1) * L, jnp.int32) + lanes
        c16 = jnp.full((L,), L, jnp.int32)
        zero = jnp.zeros((L,), jnp.int32)

        @plsc.parallel_loop(0, NVREG, unroll=8, carry=(lanes, lanes))
        def collect(i, st):
            ptr16, iv = st
            v = row_v[pl.ds(i * L, L)]
            msk = v >= t0v
            pos = jnp.minimum(ptr16, clampv)
            plsc.store_scatter(cidx, [pos], iv, mask=msk)
            return ptr16 + jnp.where(msk, c16, zero), iv + c16

        ptr16, _ = collect
        maxd = jnp.minimum(
            lax.shift_right_logical(_bfly(ptr16, jnp.maximum, lanes)[0], 4),
            CAP - 1)
        cntd = lax.shift_right_logical(ptr16 - lanes, 4)

        @plsc.parallel_loop(0, maxd + 1, carry=jnp.int32(0))
        def mat_loop(d, carry):
            dv = jnp.full((L,), d, jnp.int32)
            valid = dv < cntd
            idxv = jnp.where(valid, cidx[pl.ds(d * L, L)], zero)
            vv = plsc.load_gather(row_v, [idxv])
            cval[pl.ds(d * L, L)] = jnp.where(valid, vv, neg)
            return carry

        del mat_loop

        def isort_body(i, carry):
            kv = cval[pl.ds(i * L, L)]
            ki = cidx[pl.ds(i * L, L)]

            def down(jj, moving):
                j = i - 1 - jj
                vj = cval[pl.ds(j * L, L)]
                ij = cidx[pl.ds(j * L, L)]
                lt = (kv > vj) | ((kv == vj) & (ki < ij))
                pos1 = (j + 1) * L + lanes
                wv = jnp.where(lt, vj, kv)
                wi = jnp.where(lt, ij, ki)
                plsc.store_scatter(cval, [pos1], wv, mask=moving)
                plsc.store_scatter(cidx, [pos1], wi, mask=moving)
                return moving & lt

            moving = lax.fori_loop(0, i, down, jnp.ones((L,), jnp.bool_))
            plsc.store_scatter(cval, [lanes], kv, mask=moving)
            plsc.store_scatter(cidx, [lanes], ki, mask=moving)
            return carry

        lax.fori_loop(1, maxd, isort_body, 0)

        def merge_round(k, hp):
            addr = hp * L + lanes
            hv = plsc.load_gather(cval, [addr])
            hi = plsc.load_gather(cidx, [addr])
            mv = _bfly(hv, jnp.maximum, lanes)
            lm = hv == mv
            civ = _bfly(jnp.where(lm, hi, bigi), jnp.minimum, lanes)
            chosen = lm & (hi == civ)
            l0 = lanes == 0
            kv = jnp.full((L,), k, jnp.int32)
            plsc.store_scatter(oval, [kv], mv, mask=l0)
            plsc.store_scatter(oidx, [kv], civ, mask=l0)
            return hp + chosen.astype(jnp.int32)

        lax.fori_loop(0, TOPK, merge_round, jnp.zeros((L,), jnp.int32))

        pltpu.sync_copy(oval, vals_hbm.at[r])
        pltpu.sync_copy(oidx, idxs_hbm.at[r])


def kernel(inp, shared_refpanel):
    vals, idxs = _topk_kernel(inp)
    return (vals, idxs)
